# initial kernel scaffold (unmeasured)
import jax
import jax.numpy as jnp
from jax import lax
from jax.experimental import pallas as pl
from jax.experimental.pallas import tpu as pltpu

S = 1024
D = 2048
H = 16
DH = 128
DR = 32
DC_SH = 128
SCALE = (DH + DR) ** -0.5
F32 = jnp.float32


def _kv_body(x_ref, wdkv_ref, wuk_ref, wuv_ref, wkr_ref,
             k_ref, v_ref, kr_ref,
             c_mine, c_peer, wuk_peer, wuv_peer,
             send_sems, recv_sems):
    my_x = lax.axis_index("x")
    my_y = lax.axis_index("y")
    my_z = lax.axis_index("z")
    peer = (my_x, 1 - my_y, my_z)

    barrier_sem = pltpu.get_barrier_semaphore()
    pl.semaphore_signal(barrier_sem, inc=1, device_id=peer,
                        device_id_type=pl.DeviceIdType.MESH)
    pl.semaphore_wait(barrier_sem, 1)

    c_mine[...] = jnp.dot(x_ref[...], wdkv_ref[...], preferred_element_type=F32)

    rdma_c = pltpu.make_async_remote_copy(
        src_ref=c_mine, dst_ref=c_peer,
        send_sem=send_sems.at[0], recv_sem=recv_sems.at[0],
        device_id=peer, device_id_type=pl.DeviceIdType.MESH)
    rdma_wuk = pltpu.make_async_remote_copy(
        src_ref=wuk_ref, dst_ref=wuk_peer,
        send_sem=send_sems.at[1], recv_sem=recv_sems.at[1],
        device_id=peer, device_id_type=pl.DeviceIdType.MESH)
    rdma_wuv = pltpu.make_async_remote_copy(
        src_ref=wuv_ref, dst_ref=wuv_peer,
        send_sem=send_sems.at[2], recv_sem=recv_sems.at[2],
        device_id=peer, device_id_type=pl.DeviceIdType.MESH)
    rdma_c.start()
    rdma_wuk.start()
    rdma_wuv.start()

    kr_ref[...] = jnp.dot(x_ref[...], wkr_ref[...], preferred_element_type=F32)
    k_ref[...] = jnp.dot(c_mine[...], wuk_ref[...], preferred_element_type=F32)
    v_ref[...] = jnp.dot(c_mine[...], wuv_ref[...], preferred_element_type=F32)

    rdma_c.wait()
    rdma_wuk.wait()
    k_ref[...] += jnp.dot(c_peer[...], wuk_peer[...], preferred_element_type=F32)
    rdma_wuv.wait()
    v_ref[...] += jnp.dot(c_peer[...], wuv_peer[...], preferred_element_type=F32)


def _attn_body(x_ref, wq_ref, wqr_ref, kr_ref, k_ref, v_ref, wo_ref, out_ref):
    h = pl.program_id(0)
    q = jnp.dot(x_ref[...], wq_ref[...], preferred_element_type=F32)
    qr = jnp.dot(x_ref[...], wqr_ref[...], preferred_element_type=F32)
    scores = (jnp.dot(q, k_ref[...].T, preferred_element_type=F32)
              + jnp.dot(qr, kr_ref[...].T, preferred_element_type=F32)) * SCALE
    m = jnp.max(scores, axis=-1, keepdims=True)
    p = jnp.exp(scores - m)
    p = p / jnp.sum(p, axis=-1, keepdims=True)
    o = jnp.dot(p, v_ref[...], preferred_element_type=F32)
    contrib = jnp.dot(o, wo_ref[...], preferred_element_type=F32)

    @pl.when(h == 0)
    def _():
        out_ref[...] = jnp.zeros_like(out_ref)

    out_ref[...] += contrib


def kernel(x, Wdkv, Wuk, Wuv, Wq, Wqr, Wkr, Wo):
    x2 = x.reshape(S, D)

    k, v, kr = pl.pallas_call(
        _kv_body,
        out_shape=[
            jax.ShapeDtypeStruct((S, D), F32),
            jax.ShapeDtypeStruct((S, D), F32),
            jax.ShapeDtypeStruct((S, DR), F32),
        ],
        in_specs=[pl.BlockSpec(memory_space=pltpu.VMEM)] * 5,
        out_specs=[pl.BlockSpec(memory_space=pltpu.VMEM)] * 3,
        scratch_shapes=[
            pltpu.VMEM((S, DC_SH), F32),
            pltpu.VMEM((S, DC_SH), F32),
            pltpu.VMEM((DC_SH, D), F32),
            pltpu.VMEM((DC_SH, D), F32),
            pltpu.SemaphoreType.DMA((3,)),
            pltpu.SemaphoreType.DMA((3,)),
        ],
        compiler_params=pltpu.CompilerParams(collective_id=0),
    )(x2, Wdkv, Wuk, Wuv, Wkr)

    out = pl.pallas_call(
        _attn_body,
        grid=(H,),
        out_shape=jax.ShapeDtypeStruct((S, D), F32),
        in_specs=[
            pl.BlockSpec((S, D), lambda h: (0, 0)),
            pl.BlockSpec((D, DH), lambda h: (0, h)),
            pl.BlockSpec((D, DR), lambda h: (0, h)),
            pl.BlockSpec((S, DR), lambda h: (0, 0)),
            pl.BlockSpec((S, DH), lambda h: (0, h)),
            pl.BlockSpec((S, DH), lambda h: (0, h)),
            pl.BlockSpec((DH, D), lambda h: (h, 0)),
        ],
        out_specs=pl.BlockSpec((S, D), lambda h: (0, 0)),
        compiler_params=pltpu.CompilerParams(
            dimension_semantics=("arbitrary",),
        ),
    )(x2, Wq, Wqr, kr, k, v, Wo)

    return out.reshape(1, S, D)


# baseline (device time: 184005 ns/iter reference)
import jax
import jax.numpy as jnp
from jax import lax
from jax.experimental import pallas as pl
from jax.experimental.pallas import tpu as pltpu

S = 1024
D = 2048
H = 16
DH = 128
DR = 32
DC_SH = 128
SCALE = (DH + DR) ** -0.5
F32 = jnp.float32


def _kv_body(x_ref, wdkv_ref, wuk_ref, wuv_ref, wkr_ref, wqr_ref,
             k_ref, v_ref, kr_ref, qr3_ref,
             c_mine, c_peer, wuk_peer, wuv_peer,
             send_sems, recv_sems):
    my_x = lax.axis_index("x")
    my_y = lax.axis_index("y")
    my_z = lax.axis_index("z")
    peer = (my_x, 1 - my_y, my_z)

    barrier_sem = pltpu.get_barrier_semaphore()
    pl.semaphore_signal(barrier_sem, inc=1, device_id=peer,
                        device_id_type=pl.DeviceIdType.MESH)
    pl.semaphore_wait(barrier_sem, 1)

    c_mine[...] = jnp.dot(x_ref[...], wdkv_ref[...], preferred_element_type=F32)

    rdma_c = pltpu.make_async_remote_copy(
        src_ref=c_mine, dst_ref=c_peer,
        send_sem=send_sems.at[0], recv_sem=recv_sems.at[0],
        device_id=peer, device_id_type=pl.DeviceIdType.MESH)
    rdma_wuk = pltpu.make_async_remote_copy(
        src_ref=wuk_ref, dst_ref=wuk_peer,
        send_sem=send_sems.at[1], recv_sem=recv_sems.at[1],
        device_id=peer, device_id_type=pl.DeviceIdType.MESH)
    rdma_wuv = pltpu.make_async_remote_copy(
        src_ref=wuv_ref, dst_ref=wuv_peer,
        send_sem=send_sems.at[2], recv_sem=recv_sems.at[2],
        device_id=peer, device_id_type=pl.DeviceIdType.MESH)
    rdma_c.start()
    rdma_wuk.start()
    rdma_wuv.start()

    kr_ref[...] = jnp.dot(x_ref[...], wkr_ref[...], preferred_element_type=F32)
    qr = jnp.dot(x_ref[...], wqr_ref[...], preferred_element_type=F32)
    for h in range(H):
        qr3_ref[h] = qr[:, h * DR:(h + 1) * DR]
    k_ref[...] = jnp.dot(c_mine[...], wuk_ref[...], preferred_element_type=F32)
    v_ref[...] = jnp.dot(c_mine[...], wuv_ref[...], preferred_element_type=F32)

    rdma_c.wait()
    rdma_wuk.wait()
    k_ref[...] += jnp.dot(c_peer[...], wuk_peer[...], preferred_element_type=F32)
    rdma_wuv.wait()
    v_ref[...] += jnp.dot(c_peer[...], wuv_peer[...], preferred_element_type=F32)


def _attn_body(x_ref, wq_ref, qr3_ref, kr_ref, k_ref, v_ref, wo_ref, out_ref):
    h = pl.program_id(0)
    q = jnp.dot(x_ref[...], wq_ref[...], preferred_element_type=F32)
    qr = qr3_ref[0]
    scores = (jnp.dot(q, k_ref[...].T, preferred_element_type=F32)
              + jnp.dot(qr, kr_ref[...].T, preferred_element_type=F32)) * SCALE
    m = jnp.max(scores, axis=-1, keepdims=True)
    p = jnp.exp(scores - m)
    p = p / jnp.sum(p, axis=-1, keepdims=True)
    o = jnp.dot(p, v_ref[...], preferred_element_type=F32)
    contrib = jnp.dot(o, wo_ref[...], preferred_element_type=F32)

    @pl.when(h == 0)
    def _():
        out_ref[...] = jnp.zeros_like(out_ref)

    out_ref[...] += contrib


def kernel(x, Wdkv, Wuk, Wuv, Wq, Wqr, Wkr, Wo):
    x2 = x.reshape(S, D)

    k, v, kr, qr3 = pl.pallas_call(
        _kv_body,
        out_shape=[
            jax.ShapeDtypeStruct((S, D), F32),
            jax.ShapeDtypeStruct((S, D), F32),
            jax.ShapeDtypeStruct((S, DR), F32),
            jax.ShapeDtypeStruct((H, S, DR), F32),
        ],
        in_specs=[pl.BlockSpec(memory_space=pltpu.VMEM)] * 6,
        out_specs=[pl.BlockSpec(memory_space=pltpu.VMEM)] * 4,
        scratch_shapes=[
            pltpu.VMEM((S, DC_SH), F32),
            pltpu.VMEM((S, DC_SH), F32),
            pltpu.VMEM((DC_SH, D), F32),
            pltpu.VMEM((DC_SH, D), F32),
            pltpu.SemaphoreType.DMA((3,)),
            pltpu.SemaphoreType.DMA((3,)),
        ],
        compiler_params=pltpu.CompilerParams(collective_id=0),
    )(x2, Wdkv, Wuk, Wuv, Wkr, Wqr)

    out = pl.pallas_call(
        _attn_body,
        grid=(H,),
        out_shape=jax.ShapeDtypeStruct((S, D), F32),
        in_specs=[
            pl.BlockSpec((S, D), lambda h: (0, 0)),
            pl.BlockSpec((D, DH), lambda h: (0, h)),
            pl.BlockSpec((1, S, DR), lambda h: (h, 0, 0)),
            pl.BlockSpec((S, DR), lambda h: (0, 0)),
            pl.BlockSpec((S, DH), lambda h: (0, h)),
            pl.BlockSpec((S, DH), lambda h: (0, h)),
            pl.BlockSpec((DH, D), lambda h: (h, 0)),
        ],
        out_specs=pl.BlockSpec((S, D), lambda h: (0, 0)),
        compiler_params=pltpu.CompilerParams(
            dimension_semantics=("arbitrary",),
        ),
    )(x2, Wq, qr3, kr, k, v, Wo)

    return out.reshape(1, S, D)


# device time: 137607 ns/iter; 1.3372x vs baseline; 1.3372x over previous
import jax
import jax.numpy as jnp
from jax import lax
from jax.experimental import pallas as pl
from jax.experimental.pallas import tpu as pltpu

S = 1024
D = 2048
H = 16
DH = 128
DR = 32
DC_SH = 128
N_DEV = 8
SQ = S // N_DEV
SCALE = (DH + DR) ** -0.5
F32 = jnp.float32

_DELTAS = [(0, 0, 1), (0, 1, 0), (1, 0, 0),
           (0, 1, 1), (1, 0, 1), (1, 1, 0), (1, 1, 1)]


def _my_pos():
    my_x = lax.axis_index("x")
    my_y = lax.axis_index("y")
    my_z = lax.axis_index("z")
    return my_x, my_y, my_z


def _kv_body(x_ref, wdkv_ref, wuk_ref, wuv_ref, wkr_ref, wqr_ref,
             k_ref, v_ref, kr_ref, qr3_ref,
             c_mine, c_peer, wuk_peer, wuv_peer,
             send_sems, recv_sems):
    my_x, my_y, my_z = _my_pos()
    peer = (my_x, 1 - my_y, my_z)
    lid = my_x * 4 + my_y * 2 + my_z
    qoff = lid * SQ

    barrier_sem = pltpu.get_barrier_semaphore()
    pl.semaphore_signal(barrier_sem, inc=1, device_id=peer,
                        device_id_type=pl.DeviceIdType.MESH)
    pl.semaphore_wait(barrier_sem, 1)

    rdma_wuk = pltpu.make_async_remote_copy(
        src_ref=wuk_ref, dst_ref=wuk_peer,
        send_sem=send_sems.at[1], recv_sem=recv_sems.at[1],
        device_id=peer, device_id_type=pl.DeviceIdType.MESH)
    rdma_wuv = pltpu.make_async_remote_copy(
        src_ref=wuv_ref, dst_ref=wuv_peer,
        send_sem=send_sems.at[2], recv_sem=recv_sems.at[2],
        device_id=peer, device_id_type=pl.DeviceIdType.MESH)
    rdma_wuk.start()
    rdma_wuv.start()

    c_mine[...] = jnp.dot(x_ref[...], wdkv_ref[...], preferred_element_type=F32)
    rdma_c = pltpu.make_async_remote_copy(
        src_ref=c_mine, dst_ref=c_peer,
        send_sem=send_sems.at[0], recv_sem=recv_sems.at[0],
        device_id=peer, device_id_type=pl.DeviceIdType.MESH)
    rdma_c.start()

    kr_ref[...] = jnp.dot(x_ref[...], wkr_ref[...], preferred_element_type=F32)
    qr = jnp.dot(x_ref[pl.ds(qoff, SQ), :], wqr_ref[...],
                 preferred_element_type=F32)
    for h in range(H):
        qr3_ref[h] = qr[:, h * DR:(h + 1) * DR]
    k_ref[...] = jnp.dot(c_mine[...], wuk_ref[...], preferred_element_type=F32)
    v_ref[...] = jnp.dot(c_mine[...], wuv_ref[...], preferred_element_type=F32)

    rdma_c.wait()
    rdma_wuk.wait()
    k_ref[...] += jnp.dot(c_peer[...], wuk_peer[...], preferred_element_type=F32)
    rdma_wuv.wait()
    v_ref[...] += jnp.dot(c_peer[...], wuv_peer[...], preferred_element_type=F32)


def _attn_body(x_ref, wq_ref, qr3_ref, kr_ref, k_ref, v_ref, wo_ref, out_ref):
    h = pl.program_id(0)
    my_x, my_y, my_z = _my_pos()
    qoff = (my_x * 4 + my_y * 2 + my_z) * SQ

    q = jnp.dot(x_ref[pl.ds(qoff, SQ), :], wq_ref[...],
                preferred_element_type=F32)
    qr = qr3_ref[0]
    scores = (jnp.dot(q, k_ref[...].T, preferred_element_type=F32)
              + jnp.dot(qr, kr_ref[...].T, preferred_element_type=F32)) * SCALE
    m = jnp.max(scores, axis=-1, keepdims=True)
    p = jnp.exp(scores - m)
    p = p / jnp.sum(p, axis=-1, keepdims=True)
    o = jnp.dot(p, v_ref[...], preferred_element_type=F32)
    contrib = jnp.dot(o, wo_ref[...], preferred_element_type=F32)

    @pl.when(h == 0)
    def _():
        out_ref[...] = jnp.zeros_like(out_ref)

    out_ref[...] += contrib


def _gather_body(oq_ref, out_ref, send_sems, recv_sems):
    my_x, my_y, my_z = _my_pos()
    lid = my_x * 4 + my_y * 2 + my_z

    barrier_sem = pltpu.get_barrier_semaphore()
    for dx, dy, dz in _DELTAS:
        pl.semaphore_signal(
            barrier_sem, inc=1,
            device_id=((my_x + dx) % 2, (my_y + dy) % 2, (my_z + dz) % 2),
            device_id_type=pl.DeviceIdType.MESH)
    pl.semaphore_wait(barrier_sem, len(_DELTAS))

    rdmas = []
    for k, (dx, dy, dz) in enumerate(_DELTAS):
        peer = ((my_x + dx) % 2, (my_y + dy) % 2, (my_z + dz) % 2)
        r = pltpu.make_async_remote_copy(
            src_ref=oq_ref, dst_ref=out_ref.at[lid],
            send_sem=send_sems.at[k], recv_sem=recv_sems.at[k],
            device_id=peer, device_id_type=pl.DeviceIdType.MESH)
        r.start()
        rdmas.append(r)

    out_ref[lid] = oq_ref[...]
    for r in rdmas:
        r.wait_recv()
    for r in rdmas:
        r.wait_send()


def kernel(x, Wdkv, Wuk, Wuv, Wq, Wqr, Wkr, Wo):
    x2 = x.reshape(S, D)

    k, v, kr, qr3 = pl.pallas_call(
        _kv_body,
        out_shape=[
            jax.ShapeDtypeStruct((S, D), F32),
            jax.ShapeDtypeStruct((S, D), F32),
            jax.ShapeDtypeStruct((S, DR), F32),
            jax.ShapeDtypeStruct((H, SQ, DR), F32),
        ],
        in_specs=[pl.BlockSpec(memory_space=pltpu.VMEM)] * 6,
        out_specs=[pl.BlockSpec(memory_space=pltpu.VMEM)] * 4,
        scratch_shapes=[
            pltpu.VMEM((S, DC_SH), F32),
            pltpu.VMEM((S, DC_SH), F32),
            pltpu.VMEM((DC_SH, D), F32),
            pltpu.VMEM((DC_SH, D), F32),
            pltpu.SemaphoreType.DMA((3,)),
            pltpu.SemaphoreType.DMA((3,)),
        ],
        compiler_params=pltpu.CompilerParams(collective_id=0),
    )(x2, Wdkv, Wuk, Wuv, Wkr, Wqr)

    oq = pl.pallas_call(
        _attn_body,
        grid=(H,),
        out_shape=jax.ShapeDtypeStruct((SQ, D), F32),
        in_specs=[
            pl.BlockSpec((S, D), lambda h: (0, 0)),
            pl.BlockSpec((D, DH), lambda h: (0, h)),
            pl.BlockSpec((1, SQ, DR), lambda h: (h, 0, 0)),
            pl.BlockSpec((S, DR), lambda h: (0, 0)),
            pl.BlockSpec((S, DH), lambda h: (0, h)),
            pl.BlockSpec((S, DH), lambda h: (0, h)),
            pl.BlockSpec((DH, D), lambda h: (h, 0)),
        ],
        out_specs=pl.BlockSpec((SQ, D), lambda h: (0, 0)),
        compiler_params=pltpu.CompilerParams(
            dimension_semantics=("arbitrary",),
        ),
    )(x2, Wq, qr3, kr, k, v, Wo)

    out3 = pl.pallas_call(
        _gather_body,
        out_shape=jax.ShapeDtypeStruct((N_DEV, SQ, D), F32),
        in_specs=[pl.BlockSpec(memory_space=pltpu.VMEM)],
        out_specs=pl.BlockSpec(memory_space=pltpu.VMEM),
        scratch_shapes=[
            pltpu.SemaphoreType.DMA((len(_DELTAS),)),
            pltpu.SemaphoreType.DMA((len(_DELTAS),)),
        ],
        compiler_params=pltpu.CompilerParams(collective_id=1),
    )(oq)

    return out3.reshape(1, S, D)


# device time: 91805 ns/iter; 2.0043x vs baseline; 1.4989x over previous
import jax
import jax.numpy as jnp
from jax import lax
from jax.experimental import pallas as pl
from jax.experimental.pallas import tpu as pltpu

S = 1024
D = 2048
H = 16
DH = 128
DR = 32
DC_SH = 128
N_DEV = 8
SQ = S // N_DEV
SCALE = (DH + DR) ** -0.5
F32 = jnp.float32
BF16 = jnp.bfloat16

_DELTAS = [(0, 0, 1), (0, 1, 0), (1, 0, 0),
           (0, 1, 1), (1, 0, 1), (1, 1, 0), (1, 1, 1)]


def _my_pos():
    my_x = lax.axis_index("x")
    my_y = lax.axis_index("y")
    my_z = lax.axis_index("z")
    return my_x, my_y, my_z


def _dot(a, b):
    return jnp.dot(a, b, preferred_element_type=F32)


def _kv_body(x_ref, wdkv_ref, wuk_ref, wuv_ref, wkr_ref, wqr_ref,
             k_ref, v_ref, kr_ref, qr3_ref,
             c_mine, c_peer, wuk_mine, wuk_peer, wuv_mine, wuv_peer,
             send_sems, recv_sems):
    my_x, my_y, my_z = _my_pos()
    peer = (my_x, 1 - my_y, my_z)
    lid = my_x * 4 + my_y * 2 + my_z
    qoff = lid * SQ

    barrier_sem = pltpu.get_barrier_semaphore()
    pl.semaphore_signal(barrier_sem, inc=1, device_id=peer,
                        device_id_type=pl.DeviceIdType.MESH)
    pl.semaphore_wait(barrier_sem, 1)

    wuk_mine[...] = wuk_ref[...].astype(BF16)
    wuv_mine[...] = wuv_ref[...].astype(BF16)
    rdma_wuk = pltpu.make_async_remote_copy(
        src_ref=wuk_mine, dst_ref=wuk_peer,
        send_sem=send_sems.at[1], recv_sem=recv_sems.at[1],
        device_id=peer, device_id_type=pl.DeviceIdType.MESH)
    rdma_wuv = pltpu.make_async_remote_copy(
        src_ref=wuv_mine, dst_ref=wuv_peer,
        send_sem=send_sems.at[2], recv_sem=recv_sems.at[2],
        device_id=peer, device_id_type=pl.DeviceIdType.MESH)
    rdma_wuk.start()
    rdma_wuv.start()

    xb = x_ref[...].astype(BF16)
    c_mine[...] = _dot(xb, wdkv_ref[...].astype(BF16)).astype(BF16)
    rdma_c = pltpu.make_async_remote_copy(
        src_ref=c_mine, dst_ref=c_peer,
        send_sem=send_sems.at[0], recv_sem=recv_sems.at[0],
        device_id=peer, device_id_type=pl.DeviceIdType.MESH)
    rdma_c.start()

    kr_ref[...] = _dot(xb, wkr_ref[...].astype(BF16)).astype(BF16)
    qr = _dot(x_ref[pl.ds(qoff, SQ), :].astype(BF16), wqr_ref[...].astype(BF16))
    for h in range(H):
        qr3_ref[h] = qr[:, h * DR:(h + 1) * DR].astype(BF16)
    k_ref[...] = _dot(c_mine[...], wuk_mine[...]).astype(BF16)
    v_ref[...] = _dot(c_mine[...], wuv_mine[...]).astype(BF16)

    rdma_c.wait()
    rdma_wuk.wait()
    k_ref[...] = (k_ref[...].astype(F32)
                  + _dot(c_peer[...], wuk_peer[...])).astype(BF16)
    rdma_wuv.wait()
    v_ref[...] = (v_ref[...].astype(F32)
                  + _dot(c_peer[...], wuv_peer[...])).astype(BF16)


def _attn_body(x_ref, wq_ref, qr3_ref, kr_ref, k_ref, v_ref, wo_ref, out_ref):
    h = pl.program_id(0)
    my_x, my_y, my_z = _my_pos()
    qoff = (my_x * 4 + my_y * 2 + my_z) * SQ

    q = _dot(x_ref[pl.ds(qoff, SQ), :].astype(BF16), wq_ref[...].astype(BF16))
    kb = k_ref[...].astype(BF16)
    scores = (_dot(q.astype(BF16), kb.T)
              + _dot(qr3_ref[0], kr_ref[...].T)) * SCALE
    m = jnp.max(scores, axis=-1, keepdims=True)
    p = jnp.exp(scores - m)
    p = p / jnp.sum(p, axis=-1, keepdims=True)
    o = _dot(p.astype(BF16), v_ref[...].astype(BF16))
    contrib = _dot(o.astype(BF16), wo_ref[...].astype(BF16))

    @pl.when(h == 0)
    def _():
        out_ref[...] = jnp.zeros_like(out_ref)

    out_ref[...] += contrib


def _gather_body(oq_ref, out_ref, g_ref, sbuf, send_sems, recv_sems):
    my_x, my_y, my_z = _my_pos()
    lid = my_x * 4 + my_y * 2 + my_z
    sbuf[...] = oq_ref[...].astype(BF16)

    barrier_sem = pltpu.get_barrier_semaphore()
    for dx, dy, dz in _DELTAS:
        pl.semaphore_signal(
            barrier_sem, inc=1,
            device_id=((my_x + dx) % 2, (my_y + dy) % 2, (my_z + dz) % 2),
            device_id_type=pl.DeviceIdType.MESH)
    pl.semaphore_wait(barrier_sem, len(_DELTAS))

    rdmas = []
    for k, (dx, dy, dz) in enumerate(_DELTAS):
        peer = ((my_x + dx) % 2, (my_y + dy) % 2, (my_z + dz) % 2)
        r = pltpu.make_async_remote_copy(
            src_ref=sbuf, dst_ref=g_ref.at[lid],
            send_sem=send_sems.at[k], recv_sem=recv_sems.at[k],
            device_id=peer, device_id_type=pl.DeviceIdType.MESH)
        r.start()
        rdmas.append(r)

    g_ref[lid] = sbuf[...]
    for r in rdmas:
        r.wait_recv()
    out_ref[...] = g_ref[...].astype(F32)
    for r in rdmas:
        r.wait_send()


def kernel(x, Wdkv, Wuk, Wuv, Wq, Wqr, Wkr, Wo):
    x2 = x.reshape(S, D)

    k, v, kr, qr3 = pl.pallas_call(
        _kv_body,
        out_shape=[
            jax.ShapeDtypeStruct((S, D), BF16),
            jax.ShapeDtypeStruct((S, D), BF16),
            jax.ShapeDtypeStruct((S, DR), BF16),
            jax.ShapeDtypeStruct((H, SQ, DR), BF16),
        ],
        in_specs=[pl.BlockSpec(memory_space=pltpu.VMEM)] * 6,
        out_specs=[pl.BlockSpec(memory_space=pltpu.VMEM)] * 4,
        scratch_shapes=[
            pltpu.VMEM((S, DC_SH), BF16),
            pltpu.VMEM((S, DC_SH), BF16),
            pltpu.VMEM((DC_SH, D), BF16),
            pltpu.VMEM((DC_SH, D), BF16),
            pltpu.VMEM((DC_SH, D), BF16),
            pltpu.VMEM((DC_SH, D), BF16),
            pltpu.SemaphoreType.DMA((3,)),
            pltpu.SemaphoreType.DMA((3,)),
        ],
        compiler_params=pltpu.CompilerParams(collective_id=0),
    )(x2, Wdkv, Wuk, Wuv, Wkr, Wqr)

    oq = pl.pallas_call(
        _attn_body,
        grid=(H,),
        out_shape=jax.ShapeDtypeStruct((SQ, D), F32),
        in_specs=[
            pl.BlockSpec((S, D), lambda h: (0, 0)),
            pl.BlockSpec((D, DH), lambda h: (0, h)),
            pl.BlockSpec((1, SQ, DR), lambda h: (h, 0, 0)),
            pl.BlockSpec((S, DR), lambda h: (0, 0)),
            pl.BlockSpec((S, DH), lambda h: (0, h)),
            pl.BlockSpec((S, DH), lambda h: (0, h)),
            pl.BlockSpec((DH, D), lambda h: (h, 0)),
        ],
        out_specs=pl.BlockSpec((SQ, D), lambda h: (0, 0)),
        compiler_params=pltpu.CompilerParams(
            dimension_semantics=("arbitrary",),
        ),
    )(x2, Wq, qr3, kr, k, v, Wo)

    out = pl.pallas_call(
        _gather_body,
        out_shape=jax.ShapeDtypeStruct((N_DEV, SQ, D), F32),
        in_specs=[pl.BlockSpec(memory_space=pltpu.VMEM)],
        out_specs=pl.BlockSpec(memory_space=pltpu.VMEM),
        scratch_shapes=[
            pltpu.VMEM((N_DEV, SQ, D), BF16),
            pltpu.VMEM((SQ, D), BF16),
            pltpu.SemaphoreType.DMA((len(_DELTAS),)),
            pltpu.SemaphoreType.DMA((len(_DELTAS),)),
        ],
        compiler_params=pltpu.CompilerParams(collective_id=1),
    )(oq)

    return out.reshape(1, S, D)


# device time: 84105 ns/iter; 2.1878x vs baseline; 1.0916x over previous
import jax
import jax.numpy as jnp
from jax import lax
from jax.experimental import pallas as pl
from jax.experimental.pallas import tpu as pltpu

S = 1024
D = 2048
H = 16
DH = 128
DR = 32
DC_SH = 128
N_DEV = 8
SQ = S // N_DEV
SCALE = (DH + DR) ** -0.5
F32 = jnp.float32
BF16 = jnp.bfloat16

_DELTAS = [(0, 0, 1), (0, 1, 0), (1, 0, 0),
           (0, 1, 1), (1, 0, 1), (1, 1, 0), (1, 1, 1)]


def _my_pos():
    my_x = lax.axis_index("x")
    my_y = lax.axis_index("y")
    my_z = lax.axis_index("z")
    return my_x, my_y, my_z


def _dot(a, b):
    return jnp.dot(a, b, preferred_element_type=F32)


def _kv_body(x_ref, wdkv_ref, wuk_ref, wuv_ref, wkr_ref, wqr_ref,
             k_ref, v_ref, kr_ref, qr3_ref,
             c_mine, c_peer, wuk_mine, wuk_peer, wuv_mine, wuv_peer,
             send_sems, recv_sems):
    my_x, my_y, my_z = _my_pos()
    peer = (my_x, 1 - my_y, my_z)
    lid = my_x * 4 + my_y * 2 + my_z
    qoff = lid * SQ

    barrier_sem = pltpu.get_barrier_semaphore()
    pl.semaphore_signal(barrier_sem, inc=1, device_id=peer,
                        device_id_type=pl.DeviceIdType.MESH)
    pl.semaphore_wait(barrier_sem, 1)

    wuk_mine[...] = wuk_ref[...].astype(BF16)
    wuv_mine[...] = wuv_ref[...].astype(BF16)
    rdma_wuk = pltpu.make_async_remote_copy(
        src_ref=wuk_mine, dst_ref=wuk_peer,
        send_sem=send_sems.at[1], recv_sem=recv_sems.at[1],
        device_id=peer, device_id_type=pl.DeviceIdType.MESH)
    rdma_wuv = pltpu.make_async_remote_copy(
        src_ref=wuv_mine, dst_ref=wuv_peer,
        send_sem=send_sems.at[2], recv_sem=recv_sems.at[2],
        device_id=peer, device_id_type=pl.DeviceIdType.MESH)
    rdma_wuk.start()
    rdma_wuv.start()

    xb = x_ref[...].astype(BF16)
    c_mine[...] = _dot(xb, wdkv_ref[...].astype(BF16)).astype(BF16)
    rdma_c = pltpu.make_async_remote_copy(
        src_ref=c_mine, dst_ref=c_peer,
        send_sem=send_sems.at[0], recv_sem=recv_sems.at[0],
        device_id=peer, device_id_type=pl.DeviceIdType.MESH)
    rdma_c.start()

    kr_ref[...] = _dot(xb, wkr_ref[...].astype(BF16)).astype(BF16)
    qr = _dot(x_ref[pl.ds(qoff, SQ), :].astype(BF16), wqr_ref[...].astype(BF16))
    for h in range(H):
        qr3_ref[h] = qr[:, h * DR:(h + 1) * DR].astype(BF16)
    k_ref[...] = _dot(c_mine[...], wuk_mine[...]).astype(BF16)
    v_ref[...] = _dot(c_mine[...], wuv_mine[...]).astype(BF16)

    rdma_c.wait()
    rdma_wuk.wait()
    k_ref[...] = (k_ref[...].astype(F32)
                  + _dot(c_peer[...], wuk_peer[...])).astype(BF16)
    rdma_wuv.wait()
    v_ref[...] = (v_ref[...].astype(F32)
                  + _dot(c_peer[...], wuv_peer[...])).astype(BF16)


HB = 4


def _attn_body(x_ref, wq_ref, qr3_ref, kr_ref, k_ref, v_ref, wo_ref, out_ref):
    g = pl.program_id(0)
    my_x, my_y, my_z = _my_pos()
    qoff = (my_x * 4 + my_y * 2 + my_z) * SQ

    q4 = _dot(x_ref[pl.ds(qoff, SQ), :].astype(BF16),
              wq_ref[...].astype(BF16)).astype(BF16)
    k4 = k_ref[...]
    v4 = v_ref[...]
    kr_t = kr_ref[...].T
    os = []
    for i in range(HB):
        qi = q4[:, i * DH:(i + 1) * DH]
        ki = k4[:, i * DH:(i + 1) * DH]
        scores = (_dot(qi, ki.T) + _dot(qr3_ref[i], kr_t)) * SCALE
        m = jnp.max(scores, axis=-1, keepdims=True)
        p = jnp.exp(scores - m)
        p = p / jnp.sum(p, axis=-1, keepdims=True)
        os.append(_dot(p.astype(BF16),
                       v4[:, i * DH:(i + 1) * DH]).astype(BF16))
    contrib = _dot(jnp.concatenate(os, axis=1), wo_ref[...].astype(BF16))

    @pl.when(g == 0)
    def _():
        out_ref[...] = jnp.zeros_like(out_ref)

    out_ref[...] += contrib


def _gather_body(oq_ref, out_ref, g_ref, sbuf, send_sems, recv_sems):
    my_x, my_y, my_z = _my_pos()
    lid = my_x * 4 + my_y * 2 + my_z
    sbuf[...] = oq_ref[...].astype(BF16)

    barrier_sem = pltpu.get_barrier_semaphore()
    for dx, dy, dz in _DELTAS:
        pl.semaphore_signal(
            barrier_sem, inc=1,
            device_id=((my_x + dx) % 2, (my_y + dy) % 2, (my_z + dz) % 2),
            device_id_type=pl.DeviceIdType.MESH)
    pl.semaphore_wait(barrier_sem, len(_DELTAS))

    rdmas = []
    for k, (dx, dy, dz) in enumerate(_DELTAS):
        peer = ((my_x + dx) % 2, (my_y + dy) % 2, (my_z + dz) % 2)
        r = pltpu.make_async_remote_copy(
            src_ref=sbuf, dst_ref=g_ref.at[lid],
            send_sem=send_sems.at[k], recv_sem=recv_sems.at[k],
            device_id=peer, device_id_type=pl.DeviceIdType.MESH)
        r.start()
        rdmas.append(r)

    g_ref[lid] = sbuf[...]
    for r in rdmas:
        r.wait_recv()
    out_ref[...] = g_ref[...].astype(F32)
    for r in rdmas:
        r.wait_send()


def kernel(x, Wdkv, Wuk, Wuv, Wq, Wqr, Wkr, Wo):
    x2 = x.reshape(S, D)

    k, v, kr, qr3 = pl.pallas_call(
        _kv_body,
        out_shape=[
            jax.ShapeDtypeStruct((S, D), BF16),
            jax.ShapeDtypeStruct((S, D), BF16),
            jax.ShapeDtypeStruct((S, DR), BF16),
            jax.ShapeDtypeStruct((H, SQ, DR), BF16),
        ],
        in_specs=[pl.BlockSpec(memory_space=pltpu.VMEM)] * 6,
        out_specs=[pl.BlockSpec(memory_space=pltpu.VMEM)] * 4,
        scratch_shapes=[
            pltpu.VMEM((S, DC_SH), BF16),
            pltpu.VMEM((S, DC_SH), BF16),
            pltpu.VMEM((DC_SH, D), BF16),
            pltpu.VMEM((DC_SH, D), BF16),
            pltpu.VMEM((DC_SH, D), BF16),
            pltpu.VMEM((DC_SH, D), BF16),
            pltpu.SemaphoreType.DMA((3,)),
            pltpu.SemaphoreType.DMA((3,)),
        ],
        compiler_params=pltpu.CompilerParams(collective_id=0),
    )(x2, Wdkv, Wuk, Wuv, Wkr, Wqr)

    oq = pl.pallas_call(
        _attn_body,
        grid=(H // HB,),
        out_shape=jax.ShapeDtypeStruct((SQ, D), F32),
        in_specs=[
            pl.BlockSpec((S, D), lambda g: (0, 0)),
            pl.BlockSpec((D, HB * DH), lambda g: (0, g)),
            pl.BlockSpec((HB, SQ, DR), lambda g: (g, 0, 0)),
            pl.BlockSpec((S, DR), lambda g: (0, 0)),
            pl.BlockSpec((S, HB * DH), lambda g: (0, g)),
            pl.BlockSpec((S, HB * DH), lambda g: (0, g)),
            pl.BlockSpec((HB * DH, D), lambda g: (g, 0)),
        ],
        out_specs=pl.BlockSpec((SQ, D), lambda g: (0, 0)),
        compiler_params=pltpu.CompilerParams(
            dimension_semantics=("arbitrary",),
        ),
    )(x2, Wq, qr3, kr, k, v, Wo)

    out = pl.pallas_call(
        _gather_body,
        out_shape=jax.ShapeDtypeStruct((N_DEV, SQ, D), F32),
        in_specs=[pl.BlockSpec(memory_space=pltpu.VMEM)],
        out_specs=pl.BlockSpec(memory_space=pltpu.VMEM),
        scratch_shapes=[
            pltpu.VMEM((N_DEV, SQ, D), BF16),
            pltpu.VMEM((SQ, D), BF16),
            pltpu.SemaphoreType.DMA((len(_DELTAS),)),
            pltpu.SemaphoreType.DMA((len(_DELTAS),)),
        ],
        compiler_params=pltpu.CompilerParams(collective_id=1),
    )(oq)

    return out.reshape(1, S, D)
